# baseline (speedup 1.0000x reference)
import functools

import jax
import jax.numpy as jnp
from jax import lax
from jax.experimental import pallas as pl
from jax.experimental.pallas import tpu as pltpu
from jax.experimental.pallas import tpu_sc as plsc

_NC = 2
_NS = 16
_NW = _NC * _NS
_ND = 8
_G = 8
_K = 3
_B_TEC = 256


@functools.partial(jax.jit, static_argnums=(3, 4))
def _sc_gather(tbl, idx, sidx, n_rows, dc):
    n_scs = n_rows - _B_TEC
    b_per_c = n_scs // _NC
    n_grp = b_per_c // _G
    b_per_w = _B_TEC // _NW
    smesh = plsc.ScalarSubcoreMesh(axis_name="c", num_cores=_NC)
    vmesh = plsc.VectorSubcoreMesh(
        core_axis_name="c", subcore_axis_name="s",
        num_cores=_NC, num_subcores=_NS)

    def scs_fn(tbl_hbm, idx_hbm, sidx_hbm, out_hbm,
               idx_s, rows, gsem, ssem, sidx_v, tec0, tg0):
        base = lax.axis_index("c") * b_per_c
        pltpu.sync_copy(idx_hbm.at[pl.ds(base, b_per_c)], idx_s)

        def gather_grp(g, t):
            for u in range(_G):
                pltpu.make_async_copy(
                    tbl_hbm.at[pl.ds(idx_s[g * _G + u] * _ND, _ND)],
                    rows.at[pl.ds((t * _G + u) * _ND, _ND)], gsem[t]).start()

        def gather_wait(t):
            pltpu.make_async_copy(
                tbl_hbm.at[pl.ds(0, _G * _ND)],
                rows.at[pl.ds(t * _G * _ND, _G * _ND)], gsem[t]).wait()

        def scatter_grp(g, t):
            return pltpu.make_async_copy(
                rows.at[pl.ds(t * _G * _ND, _G * _ND)],
                out_hbm.at[pl.ds((base + g * _G) * _ND, _G * _ND)], ssem[t])

        for t in range(_K):
            gather_grp(t, t)
        for t in range(_K):
            gather_wait(t)
            scatter_grp(t, t).start()

        @pl.loop(1, n_grp // _K)
        def _(j):
            g0 = j * _K
            for t in range(_K):
                scatter_grp(0, t).wait()
                gather_grp(g0 + t, t)
            for t in range(_K):
                gather_wait(t)
                scatter_grp(g0 + t, t).start()

        for t in range(_K):
            scatter_grp(0, t).wait()

    def tec_fn(tbl_hbm, idx_hbm, sidx_hbm, out_hbm,
               idx_s, rows, gsem, ssem, sidx_v, tec0, tg0):
        wid = lax.axis_index("s") * _NC + lax.axis_index("c")
        sbase = wid * b_per_w * _ND
        obase = n_scs * _ND + sbase
        pltpu.sync_copy(sidx_hbm.at[pl.ds(sbase, b_per_w * _ND)], sidx_v)

        @pl.loop(0, b_per_w)
        def _(r):
            pltpu.async_copy(
                tbl_hbm.at[sidx_v.at[pl.ds(r * _ND, _ND)]], tec0, tg0
            ).wait()
            pltpu.sync_copy(tec0, out_hbm.at[pl.ds(obase + r * _ND, _ND)])

    f = pl.kernel(
        [scs_fn, tec_fn],
        out_type=jax.ShapeDtypeStruct((n_rows * _ND, dc), jnp.float32),
        mesh=[smesh, vmesh],
        scratch_types=[
            (pltpu.SMEM @ smesh)((b_per_c,), jnp.int32),
            pltpu.VMEM_SHARED((_K * _G * _ND, dc), jnp.float32),
            [pltpu.SemaphoreType.DMA @ smesh] * _K,
            [pltpu.SemaphoreType.DMA @ smesh] * _K,
            (pltpu.VMEM @ vmesh)((b_per_w * _ND,), jnp.int32),
            (pltpu.VMEM @ vmesh)((_ND, dc), jnp.float32),
            pltpu.SemaphoreType.DMA @ vmesh,
        ],
    )
    return f(tbl, idx, sidx)


def kernel(prefix, embedding_table):
    V, D = embedding_table.shape
    B = prefix.size
    dc = D // _ND
    idx = prefix.reshape(-1).astype(jnp.int32)
    sidx = (idx[B - _B_TEC:, None] * _ND
            + jnp.arange(_ND, dtype=jnp.int32)).reshape(-1)
    tbl = embedding_table.reshape(V * _ND, dc)
    out = _sc_gather(tbl, idx, sidx, B, dc)
    return out.reshape(*prefix.shape, D)


# speedup vs baseline: 2.6218x; 2.6218x over previous
import functools

import jax
import jax.numpy as jnp
from jax import lax
from jax.experimental import pallas as pl
from jax.experimental.pallas import tpu as pltpu
from jax.experimental.pallas import tpu_sc as plsc

_NC = 2
_G = 8
_K = 5


@functools.partial(jax.jit, static_argnums=(2, 3))
def _sc_row_copy(tbl, idx, n_rows, d):
    b_per_c = n_rows // _NC
    n_grp = b_per_c // _G
    mesh = plsc.ScalarSubcoreMesh(axis_name="c", num_cores=_NC)

    @functools.partial(
        pl.kernel,
        out_type=jax.ShapeDtypeStruct((n_rows, d), jnp.float32),
        mesh=mesh,
        scratch_types=[
            pltpu.SMEM((b_per_c,), jnp.int32),
            pltpu.VMEM_SHARED((_K * _G, d), jnp.float32),
            [pltpu.SemaphoreType.DMA] * _K,
            [pltpu.SemaphoreType.DMA] * _K,
        ],
    )
    def k(tbl_hbm, idx_hbm, out_hbm, idx_s, rows, gsem, ssem):
        base = lax.axis_index("c") * b_per_c
        pltpu.sync_copy(idx_hbm.at[pl.ds(base, b_per_c)], idx_s)

        def gather_grp(g, t):
            for u in range(_G):
                pltpu.make_async_copy(
                    tbl_hbm.at[pl.ds(idx_s[g * _G + u], 1)],
                    rows.at[pl.ds(t * _G + u, 1)], gsem[t]).start()

        def gather_wait(t):
            pltpu.make_async_copy(
                tbl_hbm.at[pl.ds(0, _G)],
                rows.at[pl.ds(t * _G, _G)], gsem[t]).wait()

        def scatter_grp(g, t):
            return pltpu.make_async_copy(
                rows.at[pl.ds(t * _G, _G)],
                out_hbm.at[pl.ds(base + g * _G, _G)], ssem[t])

        for t in range(_K):
            gather_grp(t, t)
        for t in range(_K):
            gather_wait(t)
            scatter_grp(t, t).start()

        n_loop = n_grp // _K
        n_tail = n_grp - n_loop * _K

        @pl.loop(1, n_loop)
        def _(j):
            g0 = j * _K
            for t in range(_K):
                scatter_grp(0, t).wait()
                gather_grp(g0 + t, t)
            for t in range(_K):
                gather_wait(t)
                scatter_grp(g0 + t, t).start()

        for t in range(n_tail):
            scatter_grp(0, t).wait()
            gather_grp(n_loop * _K + t, t)
        for t in range(n_tail):
            gather_wait(t)
            scatter_grp(n_loop * _K + t, t).start()
        for t in range(_K):
            scatter_grp(0, t).wait()

    return k(tbl, idx)


def kernel(prefix, embedding_table):
    V, D = embedding_table.shape
    B = prefix.size
    idx = prefix.reshape(-1).astype(jnp.int32)
    out = _sc_row_copy(embedding_table, idx, B, D)
    return out.reshape(*prefix.shape, D)


# speedup vs baseline: 2.6312x; 1.0036x over previous
import functools

import jax
import jax.numpy as jnp
from jax import lax
from jax.experimental import pallas as pl
from jax.experimental.pallas import tpu as pltpu
from jax.experimental.pallas import tpu_sc as plsc

_NC = 2
_G = 8
_K = 4


@functools.partial(jax.jit, static_argnums=(2, 3))
def _sc_row_copy(tbl, idx, n_rows, d):
    b_per_c = n_rows // _NC
    n_grp = b_per_c // _G
    mesh = plsc.ScalarSubcoreMesh(axis_name="c", num_cores=_NC)

    @functools.partial(
        pl.kernel,
        out_type=jax.ShapeDtypeStruct((n_rows, d), jnp.float32),
        mesh=mesh,
        scratch_types=[
            pltpu.SMEM((b_per_c,), jnp.int32),
            pltpu.VMEM_SHARED((_K * _G, d), jnp.float32),
            [pltpu.SemaphoreType.DMA] * _K,
            [pltpu.SemaphoreType.DMA] * _K,
        ],
    )
    def k(tbl_hbm, idx_hbm, out_hbm, idx_s, rows, gsem, ssem):
        base = lax.axis_index("c") * b_per_c
        pltpu.sync_copy(idx_hbm.at[pl.ds(base, b_per_c)], idx_s)

        def gather_grp(g, t):
            for u in range(_G):
                pltpu.make_async_copy(
                    tbl_hbm.at[pl.ds(idx_s[g * _G + u], 1)],
                    rows.at[pl.ds(t * _G + u, 1)], gsem[t]).start()

        def gather_wait(t):
            pltpu.make_async_copy(
                tbl_hbm.at[pl.ds(0, _G)],
                rows.at[pl.ds(t * _G, _G)], gsem[t]).wait()

        def scatter_grp(g, t):
            return pltpu.make_async_copy(
                rows.at[pl.ds(t * _G, _G)],
                out_hbm.at[pl.ds(base + g * _G, _G)], ssem[t])

        for t in range(_K):
            gather_grp(t, t)
        for t in range(_K):
            gather_wait(t)
            scatter_grp(t, t).start()

        n_loop = n_grp // _K
        n_tail = n_grp - n_loop * _K

        @pl.loop(1, n_loop)
        def _(j):
            g0 = j * _K
            for t in range(_K):
                scatter_grp(0, t).wait()
                gather_grp(g0 + t, t)
            for t in range(_K):
                gather_wait(t)
                scatter_grp(g0 + t, t).start()

        for t in range(n_tail):
            scatter_grp(0, t).wait()
            gather_grp(n_loop * _K + t, t)
        for t in range(n_tail):
            gather_wait(t)
            scatter_grp(n_loop * _K + t, t).start()
        for t in range(_K):
            scatter_grp(0, t).wait()

    return k(tbl, idx)


def kernel(prefix, embedding_table):
    V, D = embedding_table.shape
    B = prefix.size
    idx = prefix.reshape(-1).astype(jnp.int32)
    out = _sc_row_copy(embedding_table, idx, B, D)
    return out.reshape(*prefix.shape, D)
